# baseline (speedup 1.0000x reference)
import jax
import jax.numpy as jnp
from jax.experimental import pallas as pl
from jax.experimental.pallas import tpu as pltpu

DIM = 128
CHUNK_ROWS = (10920, 10920, 10928)
N_CHUNKS = len(CHUNK_ROWS)
CHUNK_OFFS = tuple(sum(CHUNK_ROWS[:i]) for i in range(N_CHUNKS))


def _copy_body(x_hbm, o_hbm, *bufs_and_sems):
    bufs = bufs_and_sems[:N_CHUNKS]
    in_sems, out_sems = bufs_and_sems[N_CHUNKS], bufs_and_sems[N_CHUNKS + 1]

    def read(i):
        return pltpu.make_async_copy(
            x_hbm.at[pl.ds(CHUNK_OFFS[i], CHUNK_ROWS[i])], bufs[i], in_sems.at[i]
        )

    def write(i):
        return pltpu.make_async_copy(
            bufs[i], o_hbm.at[pl.ds(CHUNK_OFFS[i], CHUNK_ROWS[i])], out_sems.at[i]
        )

    for i in range(N_CHUNKS):
        read(i).start()
    for i in range(N_CHUNKS):
        read(i).wait()
        write(i).start()
    for i in range(N_CHUNKS):
        write(i).wait()


def kernel(x):
    x = x.astype(jnp.float32)
    n = x.size // DIM
    x = x.reshape(n, DIM)
    return pl.pallas_call(
        _copy_body,
        out_shape=jax.ShapeDtypeStruct((n, DIM), jnp.float32),
        in_specs=[pl.BlockSpec(memory_space=pltpu.MemorySpace.HBM)],
        out_specs=pl.BlockSpec(memory_space=pltpu.MemorySpace.HBM),
        scratch_shapes=[
            *[pltpu.VMEM((r, DIM), jnp.float32) for r in CHUNK_ROWS],
            pltpu.SemaphoreType.DMA((N_CHUNKS,)),
            pltpu.SemaphoreType.DMA((N_CHUNKS,)),
        ],
    )(x)


# speedup vs baseline: 1.0275x; 1.0275x over previous
import jax
import jax.numpy as jnp
from jax.experimental import pallas as pl
from jax.experimental.pallas import tpu as pltpu

DIM = 128
CHUNK_ROWS = (16384, 16384)
N_CHUNKS = len(CHUNK_ROWS)
CHUNK_OFFS = tuple(sum(CHUNK_ROWS[:i]) for i in range(N_CHUNKS))


def _copy_body(x_hbm, o_hbm, *bufs_and_sems):
    bufs = bufs_and_sems[:N_CHUNKS]
    in_sems, out_sems = bufs_and_sems[N_CHUNKS], bufs_and_sems[N_CHUNKS + 1]

    def read(i):
        return pltpu.make_async_copy(
            x_hbm.at[pl.ds(CHUNK_OFFS[i], CHUNK_ROWS[i])], bufs[i], in_sems.at[i]
        )

    def write(i):
        return pltpu.make_async_copy(
            bufs[i], o_hbm.at[pl.ds(CHUNK_OFFS[i], CHUNK_ROWS[i])], out_sems.at[i]
        )

    for i in range(N_CHUNKS):
        read(i).start()
    for i in range(N_CHUNKS):
        read(i).wait()
        write(i).start()
    for i in range(N_CHUNKS):
        write(i).wait()


def kernel(x):
    x = x.astype(jnp.float32)
    n = x.size // DIM
    x = x.reshape(n, DIM)
    return pl.pallas_call(
        _copy_body,
        out_shape=jax.ShapeDtypeStruct((n, DIM), jnp.float32),
        in_specs=[pl.BlockSpec(memory_space=pltpu.MemorySpace.HBM)],
        out_specs=pl.BlockSpec(memory_space=pltpu.MemorySpace.HBM),
        scratch_shapes=[
            *[pltpu.VMEM((r, DIM), jnp.float32) for r in CHUNK_ROWS],
            pltpu.SemaphoreType.DMA((N_CHUNKS,)),
            pltpu.SemaphoreType.DMA((N_CHUNKS,)),
        ],
    )(x)
